# baseline (speedup 1.0000x reference)
import functools

import jax
import jax.numpy as jnp
from jax import lax
from jax.experimental import pallas as pl
from jax.experimental.pallas import tpu as pltpu
from jax.experimental.pallas import tpu_sc as plsc

N_NODES = 10000
N_EDGES = 320000
D_IN = 128
D_H1 = 100
D1P = 112
D_H2 = 20
D2P = 32
N_GRAPHS = 64
D_OUT = 8
EPS = 1e-5

NC, NS = 2, 16
NT = NC * NS
CH = 128
EPT = N_EDGES // NT
EPT_PAD = 10240
NCHUNK = EPT_PAD // CH
N_ACC = 10008
TRASH = 10000
NZT = 10
ROWS_PER_ZT = N_NODES // NZT


@functools.lru_cache(maxsize=None)
def _make_edge_agg(dp, nbuf):
    mesh = plsc.VectorSubcoreMesh(core_axis_name="c", subcore_axis_name="s")

    @functools.partial(
        pl.kernel,
        out_type=jax.ShapeDtypeStruct((NC * N_NODES, dp), jnp.float32),
        mesh=mesh,
        compiler_params=pltpu.CompilerParams(use_tc_tiling_on_sc=False),
        scratch_types=[
            pltpu.VMEM((NCHUNK, CH), jnp.int32),
            pltpu.VMEM((NCHUNK, CH), jnp.int32),
            pltpu.VMEM((nbuf, CH, dp), jnp.float32),
            pltpu.VMEM_SHARED((N_ACC, dp), jnp.float32),
        ] + [pltpu.SemaphoreType.DMA] * nbuf,
    )
    def agg_kernel(y_hbm, src_hbm, dst_hbm, zero_hbm, out_hbm,
                   src_v, dst_v, rows_v, acc_s, *sems):
        c = lax.axis_index("c")
        s = lax.axis_index("s")
        tid = c * NS + s
        r0 = s * ROWS_PER_ZT
        @pl.when(s < NZT)
        def _zero():
            pltpu.sync_copy(zero_hbm.at[pl.ds(r0, ROWS_PER_ZT)],
                            acc_s.at[pl.ds(r0, ROWS_PER_ZT)])
        pltpu.sync_copy(src_hbm.at[tid], src_v)
        pltpu.sync_copy(dst_hbm.at[tid], dst_v)
        plsc.subcore_barrier()

        for r in range(nbuf):
            pltpu.make_async_copy(y_hbm.at[src_v.at[r]], rows_v.at[r],
                                  sems[r]).start()

        def body(k, carry):
            for r in range(nbuf):
                i = k * nbuf + r
                pltpu.make_async_copy(y_hbm.at[src_v.at[i]], rows_v.at[r],
                                      sems[r]).wait()
                pltpu.sync_copy(rows_v.at[r], acc_s.at[dst_v.at[i]], add=True)

                @pl.when(i + nbuf < NCHUNK)
                def _prefetch():
                    pltpu.make_async_copy(y_hbm.at[src_v.at[i + nbuf]],
                                          rows_v.at[r], sems[r]).start()
            return carry

        lax.fori_loop(0, NCHUNK // nbuf, body, 0)
        plsc.subcore_barrier()

        @pl.when(s < NZT)
        def _flush():
            pltpu.sync_copy(acc_s.at[pl.ds(r0, ROWS_PER_ZT)],
                            out_hbm.at[pl.ds(c * N_NODES + r0, ROWS_PER_ZT)])

    return agg_kernel


def _tc1(x, w1p):
    def body(x_ref, w_ref, o_ref):
        y = jnp.dot(x_ref[...], w_ref[...], preferred_element_type=jnp.float32)
        col = lax.broadcasted_iota(jnp.int32, y.shape, 1)
        o_ref[...] = jnp.where(col == D_H1, 1.0, y)

    return pl.pallas_call(
        body,
        grid=(10,),
        in_specs=[pl.BlockSpec((N_NODES // 10, D_IN), lambda i: (i, 0)),
                  pl.BlockSpec((D_IN, D1P), lambda i: (0, 0))],
        out_specs=pl.BlockSpec((N_NODES // 10, D1P), lambda i: (i, 0)),
        out_shape=jax.ShapeDtypeStruct((N_NODES, D1P), jnp.float32),
    )(x, w1p)


def _tc2(parts, b1p, w2p):
    def body(p0_ref, p1_ref, b_ref, w_ref, o_ref):
        agg = p0_ref[...] + p1_ref[...]
        deg = jnp.maximum(agg[:, D_H1:D_H1 + 1], 1.0)
        h = jnp.maximum(agg / deg + b_ref[...], 0.0)
        col = lax.broadcasted_iota(jnp.int32, h.shape, 1)
        h = jnp.where(col < D_H1, h, 0.0)
        y = jnp.dot(h, w_ref[...], preferred_element_type=jnp.float32)
        col2 = lax.broadcasted_iota(jnp.int32, y.shape, 1)
        o_ref[...] = jnp.where(col2 == D_H2, 1.0, y)

    nb = N_NODES // 10
    return pl.pallas_call(
        body,
        grid=(10,),
        in_specs=[pl.BlockSpec((nb, D1P), lambda i: (i, 0)),
                  pl.BlockSpec((nb, D1P), lambda i: (i + 10, 0)),
                  pl.BlockSpec((1, D1P), lambda i: (0, 0)),
                  pl.BlockSpec((D1P, D2P), lambda i: (0, 0))],
        out_specs=pl.BlockSpec((nb, D2P), lambda i: (i, 0)),
        out_shape=jax.ShapeDtypeStruct((N_NODES, D2P), jnp.float32),
    )(parts, parts, b1p, w2p)


def _tc3(parts2, b2p, gidf, self_feat, fcW1, fcb1, fcW2, fcb2, fcW3, fcb3,
         g1, be1, g2, be2):
    def body(p0_ref, p1_ref, b2_ref, gid_ref, sf_ref, w1_ref, wb1_ref,
             w2_ref, wb2_ref, w3_ref, wb3_ref, g1_ref, be1_ref, g2_ref,
             be2_ref, o_ref):
        agg = p0_ref[...] + p1_ref[...]
        deg = jnp.maximum(agg[:, D_H2:D_H2 + 1], 1.0)
        h = jnp.maximum(agg / deg + b2_ref[...], 0.0)
        col = lax.broadcasted_iota(jnp.int32, h.shape, 1)
        h = jnp.where(col < D_H2, h, 0.0)
        gid = jnp.broadcast_to(gid_ref[...], (N_GRAPHS, N_NODES))
        grow = lax.broadcasted_iota(jnp.int32, (N_GRAPHS, N_NODES), 0)
        gmat = jnp.where(gid == grow, 1.0, 0.0)
        num = jnp.dot(gmat, h, preferred_element_type=jnp.float32)
        cnt = jnp.maximum(jnp.sum(gmat, axis=1, keepdims=True), 1.0)
        hg = num / cnt
        sf = sf_ref[...]
        w1 = w1_ref[...]
        acc = jnp.zeros((N_GRAPHS, 128), jnp.float32)
        for j in range(D_H2):
            acc = acc + hg[:, j:j + 1] * jnp.dot(
                sf, w1[j * 32:(j + 1) * 32, :],
                preferred_element_type=jnp.float32)
        o1 = acc + wb1_ref[...]
        mu = jnp.mean(o1, axis=0, keepdims=True)
        var = jnp.mean((o1 - mu) ** 2, axis=0, keepdims=True)
        a1 = jnp.maximum((o1 - mu) / jnp.sqrt(var + EPS) * g1_ref[...]
                         + be1_ref[...], 0.0)
        o2 = jnp.dot(a1, w2_ref[...], preferred_element_type=jnp.float32) + wb2_ref[...]
        mu2 = jnp.mean(o2, axis=0, keepdims=True)
        var2 = jnp.mean((o2 - mu2) ** 2, axis=0, keepdims=True)
        a2 = jnp.maximum((o2 - mu2) / jnp.sqrt(var2 + EPS) * g2_ref[...]
                         + be2_ref[...], 0.0)
        o_ref[...] = jnp.dot(a2, w3_ref[...], preferred_element_type=jnp.float32) + wb3_ref[...]

    full = lambda shape: pl.BlockSpec(shape, lambda i: (0,) * len(shape))
    return pl.pallas_call(
        body,
        grid=(1,),
        in_specs=[pl.BlockSpec((N_NODES, D2P), lambda i: (0, 0)),
                  pl.BlockSpec((N_NODES, D2P), lambda i: (1, 0)),
                  full((1, D2P)),
                  full((1, N_NODES)),
                  full((N_GRAPHS, 32)),
                  full((D_H2 * 32, 128)),
                  full((1, 128)),
                  full((128, 32)),
                  full((1, 32)),
                  full((32, D_OUT)),
                  full((1, D_OUT)),
                  full((1, 128)),
                  full((1, 128)),
                  full((1, 32)),
                  full((1, 32))],
        out_specs=full((N_GRAPHS, D_OUT)),
        out_shape=jax.ShapeDtypeStruct((N_GRAPHS, D_OUT), jnp.float32),
    )(parts2, parts2, b2p, gidf, self_feat, fcW1, fcb1, fcW2, fcb2, fcW3,
      fcb3, g1, be1, g2, be2)


def kernel(x, self_feat, W1, b1, W2, b2, fcW1, fcb1, fcW2, fcb2, fcW3, fcb3,
           g1, be1, g2, be2, edge_index, graph_ids):
    pad = EPT_PAD - EPT
    src = jnp.pad(edge_index[0].astype(jnp.int32).reshape(NT, EPT),
                  ((0, 0), (0, pad))).reshape(NT, NCHUNK, CH)
    dst = jnp.pad(edge_index[1].astype(jnp.int32).reshape(NT, EPT),
                  ((0, 0), (0, pad)),
                  constant_values=TRASH).reshape(NT, NCHUNK, CH)
    gidf = graph_ids.astype(jnp.int32).reshape(1, N_NODES)
    w1p = jnp.pad(W1, ((0, 0), (0, D1P - D_H1)))
    b1p = jnp.pad(b1, (0, D1P - D_H1)).reshape(1, D1P)
    w2p = jnp.pad(W2, ((0, D1P - D_H1), (0, D2P - D_H2)))
    b2p = jnp.pad(b2, (0, D2P - D_H2)).reshape(1, D2P)
    zero1 = jnp.zeros((N_NODES, D1P), jnp.float32)
    zero2 = jnp.zeros((N_NODES, D2P), jnp.float32)

    y1 = _tc1(x, w1p)
    agg1 = _make_edge_agg(D1P, 2)(y1, src, dst, zero1)
    y2 = _tc2(agg1, b1p, w2p)
    agg2 = _make_edge_agg(D2P, 4)(y2, src, dst, zero2)
    return _tc3(agg2, b2p, gidf, self_feat,
                fcW1, fcb1.reshape(1, 128), fcW2, fcb2.reshape(1, 32),
                fcW3, fcb3.reshape(1, D_OUT),
                g1.reshape(1, 128), be1.reshape(1, 128),
                g2.reshape(1, 32), be2.reshape(1, 32))


# speedup vs baseline: 2.5281x; 2.5281x over previous
import functools

import jax
import jax.numpy as jnp
from jax import lax
from jax.experimental import pallas as pl
from jax.experimental.pallas import tpu as pltpu
from jax.experimental.pallas import tpu_sc as plsc

N_NODES = 10000
N_EDGES = 320000
D_IN = 128
D_H1 = 100
D1P = 112
D_H2 = 20
D2P = 32
N_GRAPHS = 64
D_OUT = 8
EPS = 1e-5

NC, NS = 2, 16
NT = NC * NS
CH = 80
EPT = N_EDGES // NT
EPT_PAD = EPT
NCHUNK = EPT_PAD // CH
N_ACC = 10008
TRASH = 10000
NZT = 10
ROWS_PER_ZT = N_NODES // NZT


@functools.lru_cache(maxsize=None)
def _make_edge_agg(dp, nbuf):
    mesh = plsc.VectorSubcoreMesh(core_axis_name="c", subcore_axis_name="s")

    @functools.partial(
        pl.kernel,
        out_type=jax.ShapeDtypeStruct((NC * N_NODES, dp), jnp.float32),
        mesh=mesh,
        compiler_params=pltpu.CompilerParams(use_tc_tiling_on_sc=False),
        scratch_types=[
            pltpu.VMEM((NCHUNK, CH), jnp.int32),
            pltpu.VMEM((NCHUNK, CH), jnp.int32),
            pltpu.VMEM((nbuf, CH, dp), jnp.float32),
            pltpu.VMEM_SHARED((N_ACC, dp), jnp.float32),
        ] + [pltpu.SemaphoreType.DMA] * nbuf,
    )
    def agg_kernel(y_hbm, src_hbm, dst_hbm, zero_hbm, out_hbm,
                   src_v, dst_v, rows_v, acc_s, *sems):
        c = lax.axis_index("c")
        s = lax.axis_index("s")
        tid = c * NS + s
        r0 = s * ROWS_PER_ZT
        @pl.when(s < NZT)
        def _zero():
            pltpu.sync_copy(zero_hbm.at[pl.ds(r0, ROWS_PER_ZT)],
                            acc_s.at[pl.ds(r0, ROWS_PER_ZT)])
        pltpu.sync_copy(src_hbm.at[tid], src_v)
        pltpu.sync_copy(dst_hbm.at[tid], dst_v)
        plsc.subcore_barrier()

        for r in range(nbuf):
            pltpu.make_async_copy(y_hbm.at[src_v.at[r]], rows_v.at[r],
                                  sems[r]).start()

        def body(k, carry):
            for r in range(nbuf):
                i = k * nbuf + r
                pltpu.make_async_copy(y_hbm.at[src_v.at[i]], rows_v.at[r],
                                      sems[r]).wait()
                pltpu.sync_copy(rows_v.at[r], acc_s.at[dst_v.at[i]], add=True)

                @pl.when(i + nbuf < NCHUNK)
                def _prefetch():
                    pltpu.make_async_copy(y_hbm.at[src_v.at[i + nbuf]],
                                          rows_v.at[r], sems[r]).start()
            return carry

        lax.fori_loop(0, NCHUNK // nbuf, body, 0)
        for i in range((NCHUNK // nbuf) * nbuf, NCHUNK):
            r = i % nbuf
            pltpu.make_async_copy(y_hbm.at[src_v.at[i]], rows_v.at[r],
                                  sems[r]).wait()
            pltpu.sync_copy(rows_v.at[r], acc_s.at[dst_v.at[i]], add=True)
        plsc.subcore_barrier()

        @pl.when(s < NZT)
        def _flush():
            pltpu.sync_copy(acc_s.at[pl.ds(r0, ROWS_PER_ZT)],
                            out_hbm.at[pl.ds(c * N_NODES + r0, ROWS_PER_ZT)])

    return agg_kernel


def _tc1(x, w1p):
    def body(x_ref, w_ref, o_ref):
        y = jnp.dot(x_ref[...], w_ref[...], preferred_element_type=jnp.float32)
        col = lax.broadcasted_iota(jnp.int32, y.shape, 1)
        o_ref[...] = jnp.where(col == D_H1, 1.0, y)

    return pl.pallas_call(
        body,
        grid=(10,),
        in_specs=[pl.BlockSpec((N_NODES // 10, D_IN), lambda i: (i, 0)),
                  pl.BlockSpec((D_IN, D1P), lambda i: (0, 0))],
        out_specs=pl.BlockSpec((N_NODES // 10, D1P), lambda i: (i, 0)),
        out_shape=jax.ShapeDtypeStruct((N_NODES, D1P), jnp.float32),
    )(x, w1p)


def _tc2(parts, b1p, w2p):
    def body(p0_ref, p1_ref, b_ref, w_ref, o_ref):
        agg = p0_ref[...] + p1_ref[...]
        deg = jnp.maximum(agg[:, D_H1:D_H1 + 1], 1.0)
        h = jnp.maximum(agg / deg + b_ref[...], 0.0)
        col = lax.broadcasted_iota(jnp.int32, h.shape, 1)
        h = jnp.where(col < D_H1, h, 0.0)
        y = jnp.dot(h, w_ref[...], preferred_element_type=jnp.float32)
        col2 = lax.broadcasted_iota(jnp.int32, y.shape, 1)
        o_ref[...] = jnp.where(col2 == D_H2, 1.0, y)

    nb = N_NODES // 10
    return pl.pallas_call(
        body,
        grid=(10,),
        in_specs=[pl.BlockSpec((nb, D1P), lambda i: (i, 0)),
                  pl.BlockSpec((nb, D1P), lambda i: (i + 10, 0)),
                  pl.BlockSpec((1, D1P), lambda i: (0, 0)),
                  pl.BlockSpec((D1P, D2P), lambda i: (0, 0))],
        out_specs=pl.BlockSpec((nb, D2P), lambda i: (i, 0)),
        out_shape=jax.ShapeDtypeStruct((N_NODES, D2P), jnp.float32),
    )(parts, parts, b1p, w2p)


def _tc3(parts2, b2p, gidf, self_feat, fcW1, fcb1, fcW2, fcb2, fcW3, fcb3,
         g1, be1, g2, be2):
    def body(p0_ref, p1_ref, b2_ref, gid_ref, sf_ref, w1_ref, wb1_ref,
             w2_ref, wb2_ref, w3_ref, wb3_ref, g1_ref, be1_ref, g2_ref,
             be2_ref, o_ref):
        agg = p0_ref[...] + p1_ref[...]
        deg = jnp.maximum(agg[:, D_H2:D_H2 + 1], 1.0)
        h = jnp.maximum(agg / deg + b2_ref[...], 0.0)
        col = lax.broadcasted_iota(jnp.int32, h.shape, 1)
        h = jnp.where(col < D_H2, h, 0.0)
        gid = jnp.broadcast_to(gid_ref[...], (N_GRAPHS, N_NODES))
        grow = lax.broadcasted_iota(jnp.int32, (N_GRAPHS, N_NODES), 0)
        gmat = jnp.where(gid == grow, 1.0, 0.0)
        num = jnp.dot(gmat, h, preferred_element_type=jnp.float32)
        cnt = jnp.maximum(jnp.sum(gmat, axis=1, keepdims=True), 1.0)
        hg = num / cnt
        sf = sf_ref[...]
        w1 = w1_ref[...]
        acc = jnp.zeros((N_GRAPHS, 128), jnp.float32)
        for j in range(D_H2):
            acc = acc + hg[:, j:j + 1] * jnp.dot(
                sf, w1[j * 32:(j + 1) * 32, :],
                preferred_element_type=jnp.float32)
        o1 = acc + wb1_ref[...]
        mu = jnp.mean(o1, axis=0, keepdims=True)
        var = jnp.mean((o1 - mu) ** 2, axis=0, keepdims=True)
        a1 = jnp.maximum((o1 - mu) / jnp.sqrt(var + EPS) * g1_ref[...]
                         + be1_ref[...], 0.0)
        o2 = jnp.dot(a1, w2_ref[...], preferred_element_type=jnp.float32) + wb2_ref[...]
        mu2 = jnp.mean(o2, axis=0, keepdims=True)
        var2 = jnp.mean((o2 - mu2) ** 2, axis=0, keepdims=True)
        a2 = jnp.maximum((o2 - mu2) / jnp.sqrt(var2 + EPS) * g2_ref[...]
                         + be2_ref[...], 0.0)
        o_ref[...] = jnp.dot(a2, w3_ref[...], preferred_element_type=jnp.float32) + wb3_ref[...]

    full = lambda shape: pl.BlockSpec(shape, lambda i: (0,) * len(shape))
    return pl.pallas_call(
        body,
        grid=(1,),
        in_specs=[pl.BlockSpec((N_NODES, D2P), lambda i: (0, 0)),
                  pl.BlockSpec((N_NODES, D2P), lambda i: (1, 0)),
                  full((1, D2P)),
                  full((1, N_NODES)),
                  full((N_GRAPHS, 32)),
                  full((D_H2 * 32, 128)),
                  full((1, 128)),
                  full((128, 32)),
                  full((1, 32)),
                  full((32, D_OUT)),
                  full((1, D_OUT)),
                  full((1, 128)),
                  full((1, 128)),
                  full((1, 32)),
                  full((1, 32))],
        out_specs=full((N_GRAPHS, D_OUT)),
        out_shape=jax.ShapeDtypeStruct((N_GRAPHS, D_OUT), jnp.float32),
    )(parts2, parts2, b2p, gidf, self_feat, fcW1, fcb1, fcW2, fcb2, fcW3,
      fcb3, g1, be1, g2, be2)


def kernel(x, self_feat, W1, b1, W2, b2, fcW1, fcb1, fcW2, fcb2, fcW3, fcb3,
           g1, be1, g2, be2, edge_index, graph_ids):
    pad = EPT_PAD - EPT
    src = jnp.pad(edge_index[0].astype(jnp.int32).reshape(NT, EPT),
                  ((0, 0), (0, pad))).reshape(NT, NCHUNK, CH)
    dst = jnp.pad(edge_index[1].astype(jnp.int32).reshape(NT, EPT),
                  ((0, 0), (0, pad)),
                  constant_values=TRASH).reshape(NT, NCHUNK, CH)
    gidf = graph_ids.astype(jnp.int32).reshape(1, N_NODES)
    w1p = jnp.pad(W1, ((0, 0), (0, D1P - D_H1)))
    b1p = jnp.pad(b1, (0, D1P - D_H1)).reshape(1, D1P)
    w2p = jnp.pad(W2, ((0, D1P - D_H1), (0, D2P - D_H2)))
    b2p = jnp.pad(b2, (0, D2P - D_H2)).reshape(1, D2P)
    zero1 = jnp.zeros((N_NODES, D1P), jnp.float32)
    zero2 = jnp.zeros((N_NODES, D2P), jnp.float32)

    y1 = _tc1(x, w1p)
    agg1 = _make_edge_agg(D1P, 4)(y1, src, dst, zero1)
    y2 = _tc2(agg1, b1p, w2p)
    agg2 = _make_edge_agg(D2P, 4)(y2, src, dst, zero2)
    return _tc3(agg2, b2p, gidf, self_feat,
                fcW1, fcb1.reshape(1, 128), fcW2, fcb2.reshape(1, 32),
                fcW3, fcb3.reshape(1, D_OUT),
                g1.reshape(1, 128), be1.reshape(1, 128),
                g2.reshape(1, 32), be2.reshape(1, 32))


# speedup vs baseline: 2.6263x; 1.0389x over previous
import functools

import jax
import jax.numpy as jnp
from jax import lax
from jax.experimental import pallas as pl
from jax.experimental.pallas import tpu as pltpu
from jax.experimental.pallas import tpu_sc as plsc

N_NODES = 10000
N_EDGES = 320000
D_IN = 128
D_H1 = 100
D1P = 112
D_H2 = 20
D2P = 32
N_GRAPHS = 64
D_OUT = 8
EPS = 1e-5

NC, NS = 2, 16
NT = NC * NS
CH = 80
EPT = N_EDGES // NT
EPT_PAD = EPT
NCHUNK = EPT_PAD // CH
N_ACC = 10008
TRASH = 10000
NZT = 10
ROWS_PER_ZT = N_NODES // NZT


@functools.lru_cache(maxsize=None)
def _make_edge_agg(dp, nbuf):
    mesh = plsc.VectorSubcoreMesh(core_axis_name="c", subcore_axis_name="s")

    @functools.partial(
        pl.kernel,
        out_type=jax.ShapeDtypeStruct((NC * N_NODES, dp), jnp.float32),
        mesh=mesh,
        compiler_params=pltpu.CompilerParams(use_tc_tiling_on_sc=False),
        scratch_types=[
            pltpu.VMEM((NCHUNK, CH), jnp.int32),
            pltpu.VMEM((NCHUNK, CH), jnp.int32),
            pltpu.VMEM((nbuf, CH, dp), jnp.float32),
            pltpu.VMEM_SHARED((N_ACC, dp), jnp.float32),
        ] + [pltpu.SemaphoreType.DMA] * nbuf,
    )
    def agg_kernel(y_hbm, src_hbm, dst_hbm, zero_hbm, out_hbm,
                   src_v, dst_v, rows_v, acc_s, *sems):
        c = lax.axis_index("c")
        s = lax.axis_index("s")
        tid = c * NS + s
        r0 = s * ROWS_PER_ZT
        @pl.when(s < NZT)
        def _zero():
            pltpu.sync_copy(zero_hbm.at[pl.ds(r0, ROWS_PER_ZT)],
                            acc_s.at[pl.ds(r0, ROWS_PER_ZT)])
        pltpu.sync_copy(src_hbm.at[tid], src_v)
        pltpu.sync_copy(dst_hbm.at[tid], dst_v)
        plsc.subcore_barrier()

        for r in range(nbuf):
            pltpu.make_async_copy(y_hbm.at[src_v.at[r]], rows_v.at[r],
                                  sems[r]).start()

        def body(k, carry):
            for r in range(nbuf):
                i = k * nbuf + r
                pltpu.make_async_copy(y_hbm.at[src_v.at[i]], rows_v.at[r],
                                      sems[r]).wait()
                pltpu.sync_copy(rows_v.at[r], acc_s.at[dst_v.at[i]], add=True)

                @pl.when(i + nbuf < NCHUNK)
                def _prefetch():
                    pltpu.make_async_copy(y_hbm.at[src_v.at[i + nbuf]],
                                          rows_v.at[r], sems[r]).start()
            return carry

        lax.fori_loop(0, NCHUNK // nbuf, body, 0)
        for i in range((NCHUNK // nbuf) * nbuf, NCHUNK):
            r = i % nbuf
            pltpu.make_async_copy(y_hbm.at[src_v.at[i]], rows_v.at[r],
                                  sems[r]).wait()
            pltpu.sync_copy(rows_v.at[r], acc_s.at[dst_v.at[i]], add=True)
        plsc.subcore_barrier()

        @pl.when(s < NZT)
        def _flush():
            pltpu.sync_copy(acc_s.at[pl.ds(r0, ROWS_PER_ZT)],
                            out_hbm.at[pl.ds(c * N_NODES + r0, ROWS_PER_ZT)])

    return agg_kernel


def _tc1(x, w1p):
    def body(x_ref, w_ref, o_ref):
        y = jnp.dot(x_ref[...], w_ref[...], preferred_element_type=jnp.float32)
        col = lax.broadcasted_iota(jnp.int32, y.shape, 1)
        o_ref[...] = jnp.where(col == D_H1, 1.0, y)

    return pl.pallas_call(
        body,
        grid=(10,),
        in_specs=[pl.BlockSpec((N_NODES // 10, D_IN), lambda i: (i, 0)),
                  pl.BlockSpec((D_IN, D1P), lambda i: (0, 0))],
        out_specs=pl.BlockSpec((N_NODES // 10, D1P), lambda i: (i, 0)),
        out_shape=jax.ShapeDtypeStruct((N_NODES, D1P), jnp.float32),
    )(x, w1p)


def _tc2(parts, b1p, w2p):
    def body(p0_ref, p1_ref, b_ref, w_ref, o_ref):
        agg = p0_ref[...] + p1_ref[...]
        deg = jnp.maximum(agg[:, D_H1:D_H1 + 1], 1.0)
        h = jnp.maximum(agg / deg + b_ref[...], 0.0)
        col = lax.broadcasted_iota(jnp.int32, h.shape, 1)
        h = jnp.where(col < D_H1, h, 0.0)
        y = jnp.dot(h, w_ref[...], preferred_element_type=jnp.float32)
        col2 = lax.broadcasted_iota(jnp.int32, y.shape, 1)
        o_ref[...] = jnp.where(col2 == D_H2, 1.0, y)

    nb = N_NODES // 10
    return pl.pallas_call(
        body,
        grid=(10,),
        in_specs=[pl.BlockSpec((nb, D1P), lambda i: (i, 0)),
                  pl.BlockSpec((nb, D1P), lambda i: (i + 10, 0)),
                  pl.BlockSpec((1, D1P), lambda i: (0, 0)),
                  pl.BlockSpec((D1P, D2P), lambda i: (0, 0))],
        out_specs=pl.BlockSpec((nb, D2P), lambda i: (i, 0)),
        out_shape=jax.ShapeDtypeStruct((N_NODES, D2P), jnp.float32),
    )(parts, parts, b1p, w2p)


def _tc3(parts2, b2p, gidf, self_feat, fcW1, fcb1, fcW2, fcb2, fcW3, fcb3,
         g1, be1, g2, be2):
    def body(p0_ref, p1_ref, b2_ref, gid_ref, sf_ref, w1_ref, wb1_ref,
             w2_ref, wb2_ref, w3_ref, wb3_ref, g1_ref, be1_ref, g2_ref,
             be2_ref, o_ref):
        agg = p0_ref[...] + p1_ref[...]
        deg = jnp.maximum(agg[:, D_H2:D_H2 + 1], 1.0)
        h = jnp.maximum(agg / deg + b2_ref[...], 0.0)
        col = lax.broadcasted_iota(jnp.int32, h.shape, 1)
        h = jnp.where(col < D_H2, h, 0.0)
        gid = jnp.broadcast_to(gid_ref[...], (N_GRAPHS, N_NODES))
        grow = lax.broadcasted_iota(jnp.int32, (N_GRAPHS, N_NODES), 0)
        gmat = jnp.where(gid == grow, 1.0, 0.0)
        num = jnp.dot(gmat, h, preferred_element_type=jnp.float32)
        cnt = jnp.maximum(jnp.sum(gmat, axis=1, keepdims=True), 1.0)
        hg = num / cnt
        sf = sf_ref[...]
        w1 = w1_ref[...]
        acc = jnp.zeros((N_GRAPHS, 128), jnp.float32)
        for j in range(D_H2):
            acc = acc + hg[:, j:j + 1] * jnp.dot(
                sf, w1[j * 32:(j + 1) * 32, :],
                preferred_element_type=jnp.float32)
        o1 = acc + wb1_ref[...]
        mu = jnp.mean(o1, axis=0, keepdims=True)
        var = jnp.mean((o1 - mu) ** 2, axis=0, keepdims=True)
        a1 = jnp.maximum((o1 - mu) / jnp.sqrt(var + EPS) * g1_ref[...]
                         + be1_ref[...], 0.0)
        o2 = jnp.dot(a1, w2_ref[...], preferred_element_type=jnp.float32) + wb2_ref[...]
        mu2 = jnp.mean(o2, axis=0, keepdims=True)
        var2 = jnp.mean((o2 - mu2) ** 2, axis=0, keepdims=True)
        a2 = jnp.maximum((o2 - mu2) / jnp.sqrt(var2 + EPS) * g2_ref[...]
                         + be2_ref[...], 0.0)
        o_ref[...] = jnp.dot(a2, w3_ref[...], preferred_element_type=jnp.float32) + wb3_ref[...]

    full = lambda shape: pl.BlockSpec(shape, lambda i: (0,) * len(shape))
    return pl.pallas_call(
        body,
        grid=(1,),
        in_specs=[pl.BlockSpec((N_NODES, D2P), lambda i: (0, 0)),
                  pl.BlockSpec((N_NODES, D2P), lambda i: (1, 0)),
                  full((1, D2P)),
                  full((1, N_NODES)),
                  full((N_GRAPHS, 32)),
                  full((D_H2 * 32, 128)),
                  full((1, 128)),
                  full((128, 32)),
                  full((1, 32)),
                  full((32, D_OUT)),
                  full((1, D_OUT)),
                  full((1, 128)),
                  full((1, 128)),
                  full((1, 32)),
                  full((1, 32))],
        out_specs=full((N_GRAPHS, D_OUT)),
        out_shape=jax.ShapeDtypeStruct((N_GRAPHS, D_OUT), jnp.float32),
    )(parts2, parts2, b2p, gidf, self_feat, fcW1, fcb1, fcW2, fcb2, fcW3,
      fcb3, g1, be1, g2, be2)


def kernel(x, self_feat, W1, b1, W2, b2, fcW1, fcb1, fcW2, fcb2, fcW3, fcb3,
           g1, be1, g2, be2, edge_index, graph_ids):
    pad = EPT_PAD - EPT
    src = jnp.pad(edge_index[0].astype(jnp.int32).reshape(NT, EPT),
                  ((0, 0), (0, pad))).reshape(NT, NCHUNK, CH)
    dst = jnp.pad(edge_index[1].astype(jnp.int32).reshape(NT, EPT),
                  ((0, 0), (0, pad)),
                  constant_values=TRASH).reshape(NT, NCHUNK, CH)
    gidf = graph_ids.astype(jnp.int32).reshape(1, N_NODES)
    w1p = jnp.pad(W1, ((0, 0), (0, D1P - D_H1)))
    b1p = jnp.pad(b1, (0, D1P - D_H1)).reshape(1, D1P)
    w2p = jnp.pad(W2, ((0, D1P - D_H1), (0, D2P - D_H2)))
    b2p = jnp.pad(b2, (0, D2P - D_H2)).reshape(1, D2P)
    zero1 = jnp.zeros((N_NODES, D1P), jnp.float32)
    zero2 = jnp.zeros((N_NODES, D2P), jnp.float32)

    y1 = _tc1(x, w1p)
    agg1 = _make_edge_agg(D1P, 4)(y1, src, dst, zero1)
    y2 = _tc2(agg1, b1p, w2p)
    agg2 = _make_edge_agg(D2P, 8)(y2, src, dst, zero2)
    return _tc3(agg2, b2p, gidf, self_feat,
                fcW1, fcb1.reshape(1, 128), fcW2, fcb2.reshape(1, 32),
                fcW3, fcb3.reshape(1, D_OUT),
                g1.reshape(1, 128), be1.reshape(1, 128),
                g2.reshape(1, 32), be2.reshape(1, 32))


# speedup vs baseline: 2.6754x; 1.0187x over previous
import functools

import jax
import jax.numpy as jnp
from jax import lax
from jax.experimental import pallas as pl
from jax.experimental.pallas import tpu as pltpu
from jax.experimental.pallas import tpu_sc as plsc

N_NODES = 10000
N_EDGES = 320000
D_IN = 128
D_H1 = 100
D1P = 112
D_H2 = 20
D2P = 32
N_GRAPHS = 64
D_OUT = 8
EPS = 1e-5

NC, NS = 2, 16
NT = NC * NS
CH = 80
EPT = N_EDGES // NT
EPT_PAD = EPT
NCHUNK = EPT_PAD // CH
N_ACC = 10008
TRASH = 10000
NZT = 10
ROWS_PER_ZT = N_NODES // NZT
ZROWS = 40


@functools.lru_cache(maxsize=None)
def _make_edge_agg(dp, nbuf):
    mesh = plsc.VectorSubcoreMesh(core_axis_name="c", subcore_axis_name="s")

    @functools.partial(
        pl.kernel,
        out_type=jax.ShapeDtypeStruct((NC * N_NODES, dp), jnp.float32),
        mesh=mesh,
        compiler_params=pltpu.CompilerParams(use_tc_tiling_on_sc=False),
        scratch_types=[
            pltpu.VMEM((NCHUNK, CH), jnp.int32),
            pltpu.VMEM((NCHUNK, CH), jnp.int32),
            pltpu.VMEM((nbuf, CH, dp), jnp.float32),
            pltpu.VMEM((ZROWS, dp), jnp.float32),
            pltpu.VMEM_SHARED((N_ACC, dp), jnp.float32),
            pltpu.SemaphoreType.DMA,
        ] + [pltpu.SemaphoreType.DMA] * nbuf,
    )
    def agg_kernel(y_hbm, src_hbm, dst_hbm, out_hbm,
                   src_v, dst_v, rows_v, zbuf, acc_s, zsem, *sems):
        c = lax.axis_index("c")
        s = lax.axis_index("s")
        tid = c * NS + s
        r0 = s * ROWS_PER_ZT
        @pl.when(s < NZT)
        def _zero():
            def zrow(i, carry):
                for j in range(dp // 16):
                    zbuf[i, pl.ds(j * 16, 16)] = jnp.zeros((16,), jnp.float32)
                return carry
            lax.fori_loop(0, ZROWS, zrow, 0)
            for j in range(ROWS_PER_ZT // ZROWS):
                pltpu.make_async_copy(
                    zbuf, acc_s.at[pl.ds(r0 + j * ZROWS, ZROWS)], zsem).start()
            for j in range(ROWS_PER_ZT // ZROWS):
                pltpu.make_async_copy(
                    zbuf, acc_s.at[pl.ds(r0 + j * ZROWS, ZROWS)], zsem).wait()
        pltpu.sync_copy(src_hbm.at[tid], src_v)
        pltpu.sync_copy(dst_hbm.at[tid], dst_v)
        plsc.subcore_barrier()

        for r in range(nbuf):
            pltpu.make_async_copy(y_hbm.at[src_v.at[r]], rows_v.at[r],
                                  sems[r]).start()

        def body(k, carry):
            for r in range(nbuf):
                i = k * nbuf + r
                pltpu.make_async_copy(y_hbm.at[src_v.at[i]], rows_v.at[r],
                                      sems[r]).wait()
                pltpu.sync_copy(rows_v.at[r], acc_s.at[dst_v.at[i]], add=True)

                @pl.when(i + nbuf < NCHUNK)
                def _prefetch():
                    pltpu.make_async_copy(y_hbm.at[src_v.at[i + nbuf]],
                                          rows_v.at[r], sems[r]).start()
            return carry

        lax.fori_loop(0, NCHUNK // nbuf, body, 0)
        for i in range((NCHUNK // nbuf) * nbuf, NCHUNK):
            r = i % nbuf
            pltpu.make_async_copy(y_hbm.at[src_v.at[i]], rows_v.at[r],
                                  sems[r]).wait()
            pltpu.sync_copy(rows_v.at[r], acc_s.at[dst_v.at[i]], add=True)
        plsc.subcore_barrier()

        @pl.when(s < NZT)
        def _flush():
            pltpu.sync_copy(acc_s.at[pl.ds(r0, ROWS_PER_ZT)],
                            out_hbm.at[pl.ds(c * N_NODES + r0, ROWS_PER_ZT)])

    return agg_kernel


def _tc1(x, w1p):
    def body(x_ref, w_ref, o_ref):
        y = jnp.dot(x_ref[...], w_ref[...], preferred_element_type=jnp.float32)
        col = lax.broadcasted_iota(jnp.int32, y.shape, 1)
        o_ref[...] = jnp.where(col == D_H1, 1.0, y)

    return pl.pallas_call(
        body,
        grid=(10,),
        in_specs=[pl.BlockSpec((N_NODES // 10, D_IN), lambda i: (i, 0)),
                  pl.BlockSpec((D_IN, D1P), lambda i: (0, 0))],
        out_specs=pl.BlockSpec((N_NODES // 10, D1P), lambda i: (i, 0)),
        out_shape=jax.ShapeDtypeStruct((N_NODES, D1P), jnp.float32),
    )(x, w1p)


def _tc2(parts, b1p, w2p):
    def body(p0_ref, p1_ref, b_ref, w_ref, o_ref):
        agg = p0_ref[...] + p1_ref[...]
        deg = jnp.maximum(agg[:, D_H1:D_H1 + 1], 1.0)
        h = jnp.maximum(agg / deg + b_ref[...], 0.0)
        col = lax.broadcasted_iota(jnp.int32, h.shape, 1)
        h = jnp.where(col < D_H1, h, 0.0)
        y = jnp.dot(h, w_ref[...], preferred_element_type=jnp.float32)
        col2 = lax.broadcasted_iota(jnp.int32, y.shape, 1)
        o_ref[...] = jnp.where(col2 == D_H2, 1.0, y)

    nb = N_NODES // 10
    return pl.pallas_call(
        body,
        grid=(10,),
        in_specs=[pl.BlockSpec((nb, D1P), lambda i: (i, 0)),
                  pl.BlockSpec((nb, D1P), lambda i: (i + 10, 0)),
                  pl.BlockSpec((1, D1P), lambda i: (0, 0)),
                  pl.BlockSpec((D1P, D2P), lambda i: (0, 0))],
        out_specs=pl.BlockSpec((nb, D2P), lambda i: (i, 0)),
        out_shape=jax.ShapeDtypeStruct((N_NODES, D2P), jnp.float32),
    )(parts, parts, b1p, w2p)


def _tc3(parts2, b2p, gidf, self_feat, fcW1, fcb1, fcW2, fcb2, fcW3, fcb3,
         g1, be1, g2, be2):
    def body(p0_ref, p1_ref, b2_ref, gid_ref, sf_ref, w1_ref, wb1_ref,
             w2_ref, wb2_ref, w3_ref, wb3_ref, g1_ref, be1_ref, g2_ref,
             be2_ref, o_ref):
        agg = p0_ref[...] + p1_ref[...]
        deg = jnp.maximum(agg[:, D_H2:D_H2 + 1], 1.0)
        h = jnp.maximum(agg / deg + b2_ref[...], 0.0)
        col = lax.broadcasted_iota(jnp.int32, h.shape, 1)
        h = jnp.where(col < D_H2, h, 0.0)
        gid = jnp.broadcast_to(gid_ref[...], (N_GRAPHS, N_NODES))
        grow = lax.broadcasted_iota(jnp.int32, (N_GRAPHS, N_NODES), 0)
        gmat = jnp.where(gid == grow, 1.0, 0.0)
        num = jnp.dot(gmat, h, preferred_element_type=jnp.float32)
        cnt = jnp.maximum(jnp.sum(gmat, axis=1, keepdims=True), 1.0)
        hg = num / cnt
        sf = sf_ref[...]
        w1 = w1_ref[...]
        acc = jnp.zeros((N_GRAPHS, 128), jnp.float32)
        for j in range(D_H2):
            acc = acc + hg[:, j:j + 1] * jnp.dot(
                sf, w1[j * 32:(j + 1) * 32, :],
                preferred_element_type=jnp.float32)
        o1 = acc + wb1_ref[...]
        mu = jnp.mean(o1, axis=0, keepdims=True)
        var = jnp.mean((o1 - mu) ** 2, axis=0, keepdims=True)
        a1 = jnp.maximum((o1 - mu) / jnp.sqrt(var + EPS) * g1_ref[...]
                         + be1_ref[...], 0.0)
        o2 = jnp.dot(a1, w2_ref[...], preferred_element_type=jnp.float32) + wb2_ref[...]
        mu2 = jnp.mean(o2, axis=0, keepdims=True)
        var2 = jnp.mean((o2 - mu2) ** 2, axis=0, keepdims=True)
        a2 = jnp.maximum((o2 - mu2) / jnp.sqrt(var2 + EPS) * g2_ref[...]
                         + be2_ref[...], 0.0)
        o_ref[...] = jnp.dot(a2, w3_ref[...], preferred_element_type=jnp.float32) + wb3_ref[...]

    full = lambda shape: pl.BlockSpec(shape, lambda i: (0,) * len(shape))
    return pl.pallas_call(
        body,
        grid=(1,),
        in_specs=[pl.BlockSpec((N_NODES, D2P), lambda i: (0, 0)),
                  pl.BlockSpec((N_NODES, D2P), lambda i: (1, 0)),
                  full((1, D2P)),
                  full((1, N_NODES)),
                  full((N_GRAPHS, 32)),
                  full((D_H2 * 32, 128)),
                  full((1, 128)),
                  full((128, 32)),
                  full((1, 32)),
                  full((32, D_OUT)),
                  full((1, D_OUT)),
                  full((1, 128)),
                  full((1, 128)),
                  full((1, 32)),
                  full((1, 32))],
        out_specs=full((N_GRAPHS, D_OUT)),
        out_shape=jax.ShapeDtypeStruct((N_GRAPHS, D_OUT), jnp.float32),
    )(parts2, parts2, b2p, gidf, self_feat, fcW1, fcb1, fcW2, fcb2, fcW3,
      fcb3, g1, be1, g2, be2)


def kernel(x, self_feat, W1, b1, W2, b2, fcW1, fcb1, fcW2, fcb2, fcW3, fcb3,
           g1, be1, g2, be2, edge_index, graph_ids):
    pad = EPT_PAD - EPT
    src = jnp.pad(edge_index[0].astype(jnp.int32).reshape(NT, EPT),
                  ((0, 0), (0, pad))).reshape(NT, NCHUNK, CH)
    dst = jnp.pad(edge_index[1].astype(jnp.int32).reshape(NT, EPT),
                  ((0, 0), (0, pad)),
                  constant_values=TRASH).reshape(NT, NCHUNK, CH)
    gidf = graph_ids.astype(jnp.int32).reshape(1, N_NODES)
    w1p = jnp.pad(W1, ((0, 0), (0, D1P - D_H1)))
    b1p = jnp.pad(b1, (0, D1P - D_H1)).reshape(1, D1P)
    w2p = jnp.pad(W2, ((0, D1P - D_H1), (0, D2P - D_H2)))
    b2p = jnp.pad(b2, (0, D2P - D_H2)).reshape(1, D2P)
    y1 = _tc1(x, w1p)
    agg1 = _make_edge_agg(D1P, 4)(y1, src, dst)
    y2 = _tc2(agg1, b1p, w2p)
    agg2 = _make_edge_agg(D2P, 8)(y2, src, dst)
    return _tc3(agg2, b2p, gidf, self_feat,
                fcW1, fcb1.reshape(1, 128), fcW2, fcb2.reshape(1, 32),
                fcW3, fcb3.reshape(1, D_OUT),
                g1.reshape(1, 128), be1.reshape(1, 128),
                g2.reshape(1, 32), be2.reshape(1, 32))
